# Initial kernel scaffold; baseline (speedup 1.0000x reference)
import jax
import jax.numpy as jnp
from jax.experimental import pallas as pl


def kernel(x, dictionary, W, b):
    raise NotImplementedError("write your pallas kernel here")


# speedup vs baseline: 23.7157x; 23.7157x over previous
import jax
import jax.numpy as jnp
from jax.experimental import pallas as pl

_DIM = 32
_ATOMS = 512
_K = 8


def _body(xp_ref, xl_ref, dct_ref, dt_ref, wt_ref, b_ref,
          rep_ref, rec_ref, sq_ref, reg_ref, cnt_ref):
    first = jnp.logical_and(pl.program_id(0) == 0, pl.program_id(1) == 0)

    xb = xp_ref[...]
    xl = xl_ref[0, 0]
    dct = dct_ref[...]
    dt = dt_ref[...]
    wt = wt_ref[...]
    bias = b_ref[...]

    logits = jax.lax.dot_general(
        xb, wt, (((1,), (0,)), ((), ())),
        preferred_element_type=jnp.float32) + bias
    mx = jnp.max(logits, axis=1, keepdims=True)
    e = jnp.exp(logits - mx)
    s = jnp.sum(e, axis=1, keepdims=True)

    cross = jax.lax.dot_general(
        xb, dt, (((1,), (0,)), ((), ())),
        preferred_element_type=jnp.float32)
    dsq = jnp.sum(dt * dt, axis=0, keepdims=True)
    score = dsq - 2.0 * cross

    inf = jnp.float32(jnp.inf)
    dw = score
    for _ in range(_K):
        mn = jnp.min(dw, axis=1, keepdims=True)
        dw = jnp.where(dw == mn, inf, dw)
    mask = dw == inf

    rep = jnp.where(mask, e / s, 0.0)
    rep_ref[...] = rep

    y = jax.lax.dot_general(
        rep, dct, (((1,), (0,)), ((), ())),
        preferred_element_type=jnp.float32)
    rec_ref[...] = xl + (y - xl)

    diff = xl - y
    sq_part = jnp.sum(diff * diff, axis=(0, 1), keepdims=True)
    reg_part = jnp.sum(rep, axis=(0, 1), keepdims=True)
    cnt_part = jnp.sum(mask.astype(jnp.float32), axis=0, keepdims=True)

    @pl.when(first)
    def _():
        sq_ref[...] = jnp.zeros_like(sq_ref)
        reg_ref[...] = jnp.zeros_like(reg_ref)
        cnt_ref[...] = jnp.zeros_like(cnt_ref)

    sq_ref[...] += sq_part
    reg_ref[...] += reg_part
    cnt_ref[...] += cnt_part


def kernel(x, dictionary, W, b):
    B, C, H, Wd = x.shape
    HW = H * Wd
    N = B * HW
    R = HW // _DIM
    NB = _DIM

    xp = jnp.transpose(x.reshape(B, C, HW), (0, 2, 1)).reshape(N, _DIM)
    xls = x.reshape(B, C, R, _DIM)
    dt = dictionary.T
    wt = W.T
    br = b.reshape(1, _ATOMS)

    rep, rec, sq, reg, cnt = pl.pallas_call(
        _body,
        grid=(B, NB),
        in_specs=[
            pl.BlockSpec((R, _DIM), lambda bb, ii: (bb * NB + ii, 0)),
            pl.BlockSpec((1, 1, R, _DIM), lambda bb, ii: (bb, ii, 0, 0)),
            pl.BlockSpec((_ATOMS, _DIM), lambda bb, ii: (0, 0)),
            pl.BlockSpec((_DIM, _ATOMS), lambda bb, ii: (0, 0)),
            pl.BlockSpec((_DIM, _ATOMS), lambda bb, ii: (0, 0)),
            pl.BlockSpec((1, _ATOMS), lambda bb, ii: (0, 0)),
        ],
        out_specs=(
            pl.BlockSpec((R, _ATOMS), lambda bb, ii: (bb * NB + ii, 0)),
            pl.BlockSpec((R, _DIM), lambda bb, ii: (bb * NB + ii, 0)),
            pl.BlockSpec((1, 1), lambda bb, ii: (0, 0)),
            pl.BlockSpec((1, 1), lambda bb, ii: (0, 0)),
            pl.BlockSpec((1, _ATOMS), lambda bb, ii: (0, 0)),
        ),
        out_shape=(
            jax.ShapeDtypeStruct((N, _ATOMS), jnp.float32),
            jax.ShapeDtypeStruct((N, _DIM), jnp.float32),
            jax.ShapeDtypeStruct((1, 1), jnp.float32),
            jax.ShapeDtypeStruct((1, 1), jnp.float32),
            jax.ShapeDtypeStruct((1, _ATOMS), jnp.float32),
        ),
    )(xp, xls, dictionary, dt, wt, br)

    loss = 2.0 * sq[0, 0] / jnp.float32(N * _DIM) + reg[0, 0]
    avg_probs = cnt[0] / jnp.float32(N)
    avg_probs = avg_probs / jnp.sum(avg_probs)
    perplexity = jnp.exp(-jnp.sum(avg_probs * jnp.log(avg_probs + 1e-10)))
    reconstruction = rec.reshape(B, C, H, Wd)
    return (loss, reconstruction, perplexity, rep)


# speedup vs baseline: 27.1217x; 1.1436x over previous
import jax
import jax.numpy as jnp
from jax.experimental import pallas as pl

_DIM = 32
_ATOMS = 512
_K = 8
_CH = 56


def _body(xp_ref, xl_ref, dct_ref, wd_ref, b_ref,
          rep_ref, rec_ref, sq_ref, reg_ref, cnt_ref):
    first = jnp.logical_and(pl.program_id(0) == 0, pl.program_id(1) == 0)

    xb = xp_ref[...]
    xl = xl_ref[0, 0]
    dct = dct_ref[...]
    wd = wd_ref[...]
    bias = b_ref[...]

    dt = wd[:, _ATOMS:]
    dsq = jnp.sum(dt * dt, axis=0, keepdims=True)

    m = jax.lax.dot_general(
        xb, wd, (((1,), (0,)), ((), ())), preferred_element_type=jnp.float32)

    R = xb.shape[0]
    ch = _CH if R % _CH == 0 else R
    inf = jnp.float32(jnp.inf)
    sq_acc = jnp.zeros((1, 1), jnp.float32)
    reg_acc = jnp.zeros((1, 1), jnp.float32)
    cnt_acc = jnp.zeros((1, _ATOMS), jnp.float32)

    for r0 in range(0, R, ch):
        mc = m[r0:r0 + ch]
        score = dsq - 2.0 * mc[:, _ATOMS:]

        q = _ATOMS // 4
        s1, s2 = score[:, 0:q], score[:, q:2 * q]
        s3, s4 = score[:, 2 * q:3 * q], score[:, 3 * q:]
        a, b_ = jnp.minimum(s1, s2), jnp.maximum(s1, s2)
        c, d_ = jnp.minimum(s3, s4), jnp.maximum(s3, s4)
        t1, t4 = jnp.minimum(a, c), jnp.maximum(b_, d_)
        u, v = jnp.maximum(a, c), jnp.minimum(b_, d_)
        t2, t3 = jnp.minimum(u, v), jnp.maximum(u, v)
        for _ in range(_K):
            mn = jnp.min(t1, axis=1, keepdims=True)
            hit = t1 == mn
            t1 = jnp.where(hit, t2, t1)
            t2 = jnp.where(hit, t3, t2)
            t3 = jnp.where(hit, t4, t3)
            t4 = jnp.where(hit, inf, t4)
        mask = score <= mn

        logits = mc[:, :_ATOMS] + bias
        e = jnp.exp(logits)
        recip = 1.0 / jnp.sum(e, axis=1, keepdims=True)
        rep = jnp.where(mask, e * recip, 0.0)
        rep_ref[pl.ds(r0, ch), :] = rep

        y = jax.lax.dot_general(
            rep, dct, (((1,), (0,)), ((), ())),
            preferred_element_type=jnp.float32)
        xlc = xl[r0:r0 + ch]
        rec_ref[pl.ds(r0, ch), :] = xlc + (y - xlc)

        diff = xlc - y
        sq_acc += jnp.sum(diff * diff, axis=(0, 1), keepdims=True)
        reg_acc += jnp.sum(rep, axis=(0, 1), keepdims=True)
        cnt_acc += jnp.sum(mask.astype(jnp.float32), axis=0, keepdims=True)

    @pl.when(first)
    def _():
        sq_ref[...] = jnp.zeros_like(sq_ref)
        reg_ref[...] = jnp.zeros_like(reg_ref)
        cnt_ref[...] = jnp.zeros_like(cnt_ref)

    sq_ref[...] += sq_acc
    reg_ref[...] += reg_acc
    cnt_ref[...] += cnt_acc


def kernel(x, dictionary, W, b):
    B, C, H, Wd = x.shape
    HW = H * Wd
    N = B * HW
    R = HW // _DIM
    NB = _DIM

    xp = jnp.transpose(x.reshape(B, C, HW), (0, 2, 1)).reshape(N, _DIM)
    xls = x.reshape(B, C, R, _DIM)
    wd = jnp.concatenate([W.T, dictionary.T], axis=1)
    br = b.reshape(1, _ATOMS)

    rep, rec, sq, reg, cnt = pl.pallas_call(
        _body,
        grid=(B, NB),
        in_specs=[
            pl.BlockSpec((R, _DIM), lambda bb, ii: (bb * NB + ii, 0)),
            pl.BlockSpec((1, 1, R, _DIM), lambda bb, ii: (bb, ii, 0, 0)),
            pl.BlockSpec((_ATOMS, _DIM), lambda bb, ii: (0, 0)),
            pl.BlockSpec((_DIM, 2 * _ATOMS), lambda bb, ii: (0, 0)),
            pl.BlockSpec((1, _ATOMS), lambda bb, ii: (0, 0)),
        ],
        out_specs=(
            pl.BlockSpec((R, _ATOMS), lambda bb, ii: (bb * NB + ii, 0)),
            pl.BlockSpec((R, _DIM), lambda bb, ii: (bb * NB + ii, 0)),
            pl.BlockSpec((1, 1), lambda bb, ii: (0, 0)),
            pl.BlockSpec((1, 1), lambda bb, ii: (0, 0)),
            pl.BlockSpec((1, _ATOMS), lambda bb, ii: (0, 0)),
        ),
        out_shape=(
            jax.ShapeDtypeStruct((N, _ATOMS), jnp.float32),
            jax.ShapeDtypeStruct((N, _DIM), jnp.float32),
            jax.ShapeDtypeStruct((1, 1), jnp.float32),
            jax.ShapeDtypeStruct((1, 1), jnp.float32),
            jax.ShapeDtypeStruct((1, _ATOMS), jnp.float32),
        ),
    )(xp, xls, dictionary, wd, br)

    loss = 2.0 * sq[0, 0] / jnp.float32(N * _DIM) + reg[0, 0]
    avg_probs = cnt[0] / jnp.float32(N)
    avg_probs = avg_probs / jnp.sum(avg_probs)
    perplexity = jnp.exp(-jnp.sum(avg_probs * jnp.log(avg_probs + 1e-10)))
    reconstruction = rec.reshape(B, C, H, Wd)
    return (loss, reconstruction, perplexity, rep)
